# baseline (speedup 1.0000x reference)
import functools

import jax
import jax.numpy as jnp
from jax import lax
from jax.experimental import pallas as pl
from jax.experimental.pallas import tpu as pltpu
from jax.experimental.pallas import tpu_sc as plsc

EMBED = 64
FACTOR = 8.0

NUM_CORES = 2
NUM_SUBCORES = 16
NUM_WORKERS = NUM_CORES * NUM_SUBCORES
LANES = 16
VB = 128

LINEAR = pltpu.CompilerParams(
    use_tc_tiling_on_sc=False, needs_layout_passes=False
)


@jax.jit
def _gather_out(srcT, table):
    row_len, n_rows = srcT.shape
    assert n_rows == NUM_WORKERS * VB and row_len % 2 == 0
    mesh = plsc.VectorSubcoreMesh(core_axis_name="c", subcore_axis_name="s")

    @functools.partial(
        pl.kernel,
        out_type=jax.ShapeDtypeStruct(
            (row_len, EMBED // 8, n_rows // VB, 8, VB), jnp.float32),
        mesh=mesh,
        scratch_types=[
            pltpu.VMEM((row_len, VB), jnp.int32),
            pltpu.VMEM((VB, EMBED), jnp.float32),
            pltpu.VMEM((VB, EMBED), jnp.float32),
            pltpu.VMEM((EMBED // 8, 8, VB + 1), jnp.float32),
            pltpu.VMEM((EMBED // 8, 8, VB + 1), jnp.float32),
            pltpu.SemaphoreType.DMA,
            pltpu.SemaphoreType.DMA,
            pltpu.SemaphoreType.DMA,
            pltpu.SemaphoreType.DMA,
            pltpu.SemaphoreType.DMA,
        ],
        compiler_params=LINEAR,
    )
    def body(table_hbm, srcT_hbm, out4_hbm, idxT, g0, g1, w0, w1,
             isem, gs0, gs1, ws0, ws1):
        wid = lax.axis_index("s") * NUM_CORES + lax.axis_index("c")
        col0 = wid * VB
        iota = lax.iota(jnp.int32, LANES)

        pltpu.async_copy(srcT_hbm.at[:, pl.ds(col0, VB)], idxT, isem)
        pltpu.make_async_copy(
            srcT_hbm.at[:, pl.ds(0, VB)], idxT, isem
        ).wait()

        r1 = [lax.shift_right_logical(iota + 16 * k, 3) for k in range(4)]
        r2 = [lax.bitwise_and(iota + 16 * k, 7) for k in range(4)]

        def fire_gather(c, gbuf, gsem):
            pltpu.async_copy(table_hbm.at[idxT.at[c]], gbuf, gsem)

        def drain_gather(gbuf, gsem):
            pltpu.make_async_copy(
                table_hbm.at[pl.ds(0, VB)], gbuf, gsem
            ).wait()

        def build(c, gbuf, wbuf):
            def step_t(t, c2):
                colt = jnp.full((LANES,), 16 * t, jnp.int32)
                for j in range(LANES):
                    b = 16 * t + j
                    vs = [
                        gbuf[b, pl.ds(16 * k, 16)]
                        for k in range(EMBED // LANES)
                    ]
                    colv = colt + j
                    for k in range(EMBED // LANES):
                        plsc.store_scatter(
                            wbuf, (r1[k], r2[k], colv), vs[k] * FACTOR
                        )
                return c2

            lax.fori_loop(0, 8, step_t, 0)

        def fire_write(c, wbuf, wsem):
            pltpu.async_copy(
                wbuf.at[:, :, pl.ds(0, VB)], out4_hbm.at[c, :, wid], wsem
            )

        def drain_write(wbuf, wsem):
            pltpu.make_async_copy(
                wbuf.at[:, :, pl.ds(0, VB)], out4_hbm.at[0, :, 0], wsem
            ).wait()

        fire_gather(0, g0, gs0)

        def step(j, carry):
            c0 = 2 * j
            c1 = c0 + 1

            drain_gather(g0, gs0)
            fire_gather(c1, g1, gs1)

            @pl.when(j > 0)
            def _():
                drain_write(w0, ws0)

            build(c0, g0, w0)
            fire_write(c0, w0, ws0)

            drain_gather(g1, gs1)

            @pl.when(c1 + 1 < row_len)
            def _():
                fire_gather(c1 + 1, g0, gs0)

            @pl.when(j > 0)
            def _():
                drain_write(w1, ws1)

            build(c1, g1, w1)
            fire_write(c1, w1, ws1)
            return carry

        lax.fori_loop(0, row_len // 2, step, 0)
        drain_write(w0, ws0)
        drain_write(w1, ws1)

    return body(table, srcT)


def kernel(src, table):
    n_rows, row_len = src.shape
    vocab, embed = table.shape
    assert embed == EMBED and n_rows == NUM_WORKERS * VB
    n_chunks = 4
    step = vocab // n_chunks
    chunks = [
        jax.lax.optimization_barrier(
            lax.slice(table, (i * step, 0), ((i + 1) * step, embed))
        )
        for i in range(n_chunks)
    ]
    tableL = jnp.concatenate(chunks, axis=0)
    out4 = _gather_out(src.T, tableL)
    return jnp.reshape(
        jnp.transpose(out4, (2, 4, 0, 1, 3)), (n_rows, row_len, embed)
    )


# speedup vs baseline: 1.7700x; 1.7700x over previous
import functools

import jax
import jax.numpy as jnp
from jax import lax
from jax.experimental import pallas as pl
from jax.experimental.pallas import tpu as pltpu
from jax.experimental.pallas import tpu_sc as plsc

EMBED = 64
FACTOR = 8.0

NUM_CORES = 2
NUM_SUBCORES = 16
NUM_WORKERS = NUM_CORES * NUM_SUBCORES
LANES = 16
VB = 128

LINEAR = pltpu.CompilerParams(
    use_tc_tiling_on_sc=False, needs_layout_passes=False
)


@jax.jit
def _gather_out(srcT, table):
    row_len, n_rows = srcT.shape
    assert n_rows == NUM_WORKERS * VB and row_len % 2 == 0
    mesh = plsc.VectorSubcoreMesh(core_axis_name="c", subcore_axis_name="s")

    @functools.partial(
        pl.kernel,
        out_type=jax.ShapeDtypeStruct(
            (row_len, EMBED // 8, n_rows // VB, 8, VB), jnp.float32),
        mesh=mesh,
        scratch_types=[
            pltpu.VMEM((row_len, VB), jnp.int32),
            pltpu.VMEM((VB, EMBED), jnp.float32),
            pltpu.VMEM((VB, EMBED), jnp.float32),
            pltpu.VMEM((EMBED // 8, 8, VB + 1), jnp.float32),
            pltpu.VMEM((EMBED // 8, 8, VB + 1), jnp.float32),
            pltpu.SemaphoreType.DMA,
            pltpu.SemaphoreType.DMA,
            pltpu.SemaphoreType.DMA,
            pltpu.SemaphoreType.DMA,
            pltpu.SemaphoreType.DMA,
        ],
        compiler_params=LINEAR,
    )
    def body(table_hbm, srcT_hbm, out4_hbm, idxT, g0, g1, w0, w1,
             isem, gs0, gs1, ws0, ws1):
        wid = lax.axis_index("s") * NUM_CORES + lax.axis_index("c")
        col0 = wid * VB
        iota = lax.iota(jnp.int32, LANES)

        pltpu.async_copy(srcT_hbm.at[:, pl.ds(col0, VB)], idxT, isem)
        pltpu.make_async_copy(
            srcT_hbm.at[:, pl.ds(0, VB)], idxT, isem
        ).wait()

        r1 = [lax.shift_right_logical(iota + 16 * k, 3) for k in range(4)]
        r2 = [lax.bitwise_and(iota + 16 * k, 7) for k in range(4)]

        def fire_gather(c, gbuf, gsem):
            pltpu.async_copy(table_hbm.at[idxT.at[c]], gbuf, gsem)

        def drain_gather(gbuf, gsem):
            pltpu.make_async_copy(
                table_hbm.at[pl.ds(0, VB)], gbuf, gsem
            ).wait()

        def build(c, gbuf, wbuf):
            def step_t(t, c2):
                colt = jnp.full((LANES,), 16 * t, jnp.int32)
                for j in range(LANES):
                    b = 16 * t + j
                    vs = [
                        gbuf[b, pl.ds(16 * k, 16)]
                        for k in range(EMBED // LANES)
                    ]
                    colv = colt + j
                    for k in range(EMBED // LANES):
                        plsc.store_scatter(
                            wbuf, (r1[k], r2[k], colv), vs[k] * FACTOR
                        )
                return c2

            lax.fori_loop(0, 8, step_t, 0)

        def fire_write(c, wbuf, wsem):
            pltpu.async_copy(
                wbuf.at[:, :, pl.ds(0, VB)], out4_hbm.at[c, :, wid], wsem
            )

        def drain_write(wbuf, wsem):
            pltpu.make_async_copy(
                wbuf.at[:, :, pl.ds(0, VB)], out4_hbm.at[0, :, 0], wsem
            ).wait()

        fire_gather(0, g0, gs0)

        def step(j, carry):
            c0 = 2 * j
            c1 = c0 + 1

            drain_gather(g0, gs0)
            fire_gather(c1, g1, gs1)

            @pl.when(j > 0)
            def _():
                drain_write(w0, ws0)

            build(c0, g0, w0)
            fire_write(c0, w0, ws0)

            drain_gather(g1, gs1)

            @pl.when(c1 + 1 < row_len)
            def _():
                fire_gather(c1 + 1, g0, gs0)

            @pl.when(j > 0)
            def _():
                drain_write(w1, ws1)

            build(c1, g1, w1)
            fire_write(c1, w1, ws1)
            return carry

        lax.fori_loop(0, row_len // 2, step, 0)
        drain_write(w0, ws0)
        drain_write(w1, ws1)

    return body(table, srcT)


def kernel(src, table):
    n_rows, row_len = src.shape
    vocab, embed = table.shape
    assert embed == EMBED and n_rows == NUM_WORKERS * VB
    out4 = _gather_out(src.T, table)
    return jnp.reshape(
        jnp.transpose(out4, (2, 4, 0, 1, 3)), (n_rows, row_len, embed)
    )
